# baseline (speedup 1.0000x reference)
import functools

import jax
import jax.numpy as jnp
from jax import lax
from jax.experimental import pallas as pl
from jax.experimental.pallas import tpu as pltpu
from jax.experimental.pallas import tpu_sc as plsc

_SCALE = 10.0
_D = 128
_S = 26
_V = 4096
_B = _V * _S
_NBUF = 6


def _make_kernel():
    info = plsc.get_sparse_core_info()
    nc, ns = info.num_cores, info.num_subcores
    nw = nc * ns
    v_per_w = _V // nw

    mesh = plsc.VectorSubcoreMesh(core_axis_name="c", subcore_axis_name="s")

    @functools.partial(
        pl.kernel,
        mesh=mesh,
        out_type=jax.ShapeDtypeStruct((_B, _D), jnp.float32),
        scratch_types=(
            [pltpu.VMEM((_S, v_per_w), jnp.int32)]
            + [pltpu.VMEM((v_per_w, _D), jnp.float32)] * _NBUF
            + [pltpu.SemaphoreType.DMA] * (2 * _NBUF)
        ),
    )
    def k(xt_hbm, w_hbm, out_hbm, idx_v, *bufs_and_sems):
        bufs = bufs_and_sems[:_NBUF]
        gsems = bufs_and_sems[_NBUF:2 * _NBUF]
        ssems = bufs_and_sems[2 * _NBUF:]
        wid = lax.axis_index("s") * nc + lax.axis_index("c")
        vbase = wid * v_per_w
        pltpu.sync_copy(xt_hbm.at[:, pl.ds(vbase, v_per_w)], idx_v)

        def gather(c, b):
            pltpu.async_copy(w_hbm.at[idx_v.at[c]], bufs[b], gsems[b])

        def out_slice(c):
            return out_hbm.at[pl.ds(c * _V + vbase, v_per_w)]

        for b in range(_NBUF):
            gather(b, b)

        def step(g, carry):
            for b in range(_NBUF):
                c = _NBUF * g + b
                buf, gs, ss = bufs[b], gsems[b], ssems[b]

                @pl.when(c < _S)
                def _():
                    pltpu.make_async_copy(
                        w_hbm.at[idx_v.at[c]], buf, gs).wait()

                    @plsc.parallel_loop(0, v_per_w, step=2, unroll=2)
                    def _(i):
                        for r in range(2):
                            for j in range(_D // 16):
                                sl = pl.ds(j * 16, 16)
                                buf[i + r, sl] = buf[i + r, sl] * _SCALE

                    pltpu.async_copy(buf, out_slice(c), ss)

                    @pl.when(c + _NBUF < _S)
                    def _():
                        pltpu.make_async_copy(buf, out_slice(c), ss).wait()
                        gather(c + _NBUF, b)
            return carry

        nsteps = (_S + _NBUF - 1) // _NBUF
        lax.fori_loop(0, nsteps, step, 0, unroll=False)

        for b in range(_NBUF):
            c = _S - _NBUF + b
            pltpu.make_async_copy(bufs[b], out_slice(c), ssems[b]).wait()

    return k


_kernel_call = _make_kernel()


def kernel(x, weight):
    x_t = x.astype(jnp.int32).T
    out = _kernel_call(x_t, weight)
    return out.reshape(_S, _V, _D).transpose(1, 0, 2)
